# Initial kernel scaffold; baseline (speedup 1.0000x reference)
import functools

import jax
import jax.numpy as jnp
from jax import lax
from jax.experimental import pallas as pl
from jax.experimental.pallas import tpu as pltpu
from jax.experimental.pallas import tpu_sc as plsc

D = 32
L = 16
NC, NS = 2, 16
NW = NC * NS
R = 64

K_GENRE, K_DIR, K_ACTOR = 8, 5, 20
P_RATE = 0
P_GENRE = 1
P_DIR = P_GENRE + K_GENRE
P_ACTOR = P_DIR + K_DIR
NP = P_ACTOR + K_ACTOR
POOLED = ((0, P_GENRE, K_GENRE), (1, P_DIR, K_DIR), (2, P_ACTOR, K_ACTOR))


@functools.lru_cache(maxsize=None)
def _build(B: int):
  rows_per_w = B // NW
  n_chunks = rows_per_w // R
  mesh = plsc.VectorSubcoreMesh(core_axis_name="c", subcore_axis_name="s")

  @functools.partial(
      pl.kernel,
      out_type=jax.ShapeDtypeStruct((B, 4 * D), jnp.float32),
      mesh=mesh,
      scratch_types=[
          pltpu.VMEM((NP, R, D), jnp.float32),
          pltpu.VMEM((NP, R), jnp.int32),
          pltpu.VMEM((2, 3, R), jnp.float32),
          pltpu.VMEM((3, D), jnp.float32),
          pltpu.VMEM((R, 4 * D), jnp.float32),
          pltpu.SemaphoreType.DMA,
      ],
  )
  def sc_kernel(idx_hbm, tr_hbm, tg_hbm, td_hbm, ta_hbm, out_hbm,
                gbuf, idxb, ab, t0, outb, sem):
    wid = lax.axis_index("c") * NS + lax.axis_index("s")

    pltpu.sync_copy(tg_hbm.at[pl.ds(0, 1)], t0.at[pl.ds(0, 1)])
    pltpu.sync_copy(td_hbm.at[pl.ds(0, 1)], t0.at[pl.ds(1, 1)])
    pltpu.sync_copy(ta_hbm.at[pl.ds(0, 1)], t0.at[pl.ds(2, 1)])
    t0v = [[t0[fi, pl.ds(h * L, L)] for h in range(2)] for fi in range(3)]

    def chunk(c, carry):
      base = wid * rows_per_w + c * R

      pltpu.sync_copy(idx_hbm.at[:, pl.ds(base, R)], idxb)

      plan = ([(P_RATE, tr_hbm)]
              + [(P_GENRE + j, tg_hbm) for j in range(K_GENRE)]
              + [(P_DIR + j, td_hbm) for j in range(K_DIR)]
              + [(P_ACTOR + j, ta_hbm) for j in range(K_ACTOR)])
      descs = [pltpu.async_copy(tbl.at[idxb.at[p]], gbuf.at[p], sem)
               for p, tbl in plan]

      for fi, p0, k in POOLED:
        for g in range(R // L):
          sl = pl.ds(g * L, L)
          cnt = jnp.zeros((L,), jnp.float32)
          for j in range(k):
            cnt = cnt + jnp.where(idxb[p0 + j, sl] > 0,
                                  jnp.float32(1.0), jnp.float32(0.0))
          a = jnp.float32(1.0) / (cnt + jnp.float32(1e-8))
          ab[0, fi, sl] = a
          ab[1, fi, sl] = (jnp.float32(k) - cnt) * a

      for d_ in descs:
        d_.wait()

      def row(r, carry2):
        for h in range(2):
          sl = pl.ds(h * L, L)
          outb[r, pl.ds(h * L, L)] = gbuf[P_RATE, r, sl]
        for fi, p0, k in POOLED:
          a = ab[0, fi, r]
          b = ab[1, fi, r]
          for h in range(2):
            sl = pl.ds(h * L, L)
            acc = gbuf[p0, r, sl]
            for j in range(1, k):
              acc = acc + gbuf[p0 + j, r, sl]
            outb[r, pl.ds((fi + 1) * D + h * L, L)] = acc * a - t0v[fi][h] * b
        return carry2

      lax.fori_loop(0, R, row, 0)

      pltpu.sync_copy(outb, out_hbm.at[pl.ds(base, R)])
      return carry

    lax.fori_loop(0, n_chunks, chunk, 0)

  return sc_kernel


def kernel(rate_idx, genre_idx, director_idx, actors_idx,
           table_rate, table_genre, table_director, table_actor):
  B = rate_idx.shape[0]
  idx_all = jnp.concatenate(
      [rate_idx[None, :], genre_idx.T, director_idx.T, actors_idx.T],
      axis=0).astype(jnp.int32)
  return _build(B)(idx_all, table_rate, table_genre, table_director,
                   table_actor)


# speedup vs baseline: 1.9617x; 1.9617x over previous
import functools

import jax
import jax.numpy as jnp
from jax import lax
from jax.experimental import pallas as pl
from jax.experimental.pallas import tpu as pltpu
from jax.experimental.pallas import tpu_sc as plsc

D = 32
L = 16
NC, NS = 2, 16
NW = NC * NS
R = 64

K_GENRE, K_DIR, K_ACTOR = 8, 5, 20
P_RATE = 0
P_GENRE = 1
P_DIR = P_GENRE + K_GENRE
P_ACTOR = P_DIR + K_DIR
NP = P_ACTOR + K_ACTOR
POOLED = ((0, P_GENRE, K_GENRE), (1, P_DIR, K_DIR), (2, P_ACTOR, K_ACTOR))


@functools.lru_cache(maxsize=None)
def _build(B: int):
  rows_per_w = B // NW
  n_chunks = rows_per_w // R
  mesh = plsc.VectorSubcoreMesh(core_axis_name="c", subcore_axis_name="s",
                                num_cores=NC, num_subcores=NS)

  @functools.partial(
      pl.kernel,
      out_type=jax.ShapeDtypeStruct((B, 4 * D), jnp.float32),
      mesh=mesh,
      compiler_params=pltpu.CompilerParams(use_tc_tiling_on_sc=False),
      scratch_types=[
          pltpu.VMEM((NP, R, D), jnp.float32),
          pltpu.VMEM((NP, R), jnp.int32),
          pltpu.VMEM((2, 3, R), jnp.float32),
          pltpu.VMEM((3, D), jnp.float32),
          pltpu.VMEM((R, 4 * D), jnp.float32),
          pltpu.SemaphoreType.DMA,
      ],
  )
  def sc_kernel(idx_hbm, tr_hbm, tg_hbm, td_hbm, ta_hbm, out_hbm,
                gbuf, idxb, ab, t0, outb, sem):
    wid = lax.axis_index("c") * NS + lax.axis_index("s")

    pltpu.sync_copy(tg_hbm.at[pl.ds(0, 1)], t0.at[pl.ds(0, 1)])
    pltpu.sync_copy(td_hbm.at[pl.ds(0, 1)], t0.at[pl.ds(1, 1)])
    pltpu.sync_copy(ta_hbm.at[pl.ds(0, 1)], t0.at[pl.ds(2, 1)])
    t0v = [[t0[fi, pl.ds(h * L, L)] for h in range(2)] for fi in range(3)]

    def chunk(c, carry):
      base = wid * rows_per_w + c * R

      pltpu.sync_copy(idx_hbm.at[wid, c], idxb)

      plan = ([(P_RATE, tr_hbm)]
              + [(P_GENRE + j, tg_hbm) for j in range(K_GENRE)]
              + [(P_DIR + j, td_hbm) for j in range(K_DIR)]
              + [(P_ACTOR + j, ta_hbm) for j in range(K_ACTOR)])
      descs = [pltpu.async_copy(tbl.at[idxb.at[p]], gbuf.at[p], sem)
               for p, tbl in plan]

      for fi, p0, k in POOLED:
        for g in range(R // L):
          sl = pl.ds(g * L, L)
          cnt = jnp.zeros((L,), jnp.float32)
          for j in range(k):
            cnt = cnt + jnp.where(idxb[p0 + j, sl] > 0,
                                  jnp.float32(1.0), jnp.float32(0.0))
          a = jnp.float32(1.0) / (cnt + jnp.float32(1e-8))
          ab[0, fi, sl] = a
          ab[1, fi, sl] = (jnp.float32(k) - cnt) * a

      for d_ in descs:
        d_.wait()

      def rowgrp(g, carry2):
        gsl = pl.ds(g * L, L)
        av = [ab[0, fi, gsl] for fi in range(3)]
        bv = [ab[1, fi, gsl] for fi in range(3)]
        for lane in range(L):
          r = g * L + lane
          for h in range(2):
            sl = pl.ds(h * L, L)
            outb[r, pl.ds(h * L, L)] = gbuf[P_RATE, r, sl]
          for fi, p0, k in POOLED:
            a = av[fi][lane]
            b = bv[fi][lane]
            for h in range(2):
              sl = pl.ds(h * L, L)
              acc = gbuf[p0, r, sl]
              for j in range(1, k):
                acc = acc + gbuf[p0 + j, r, sl]
              outb[r, pl.ds((fi + 1) * D + h * L, L)] = (
                  acc * a - t0v[fi][h] * b)
        return carry2

      lax.fori_loop(0, R // L, rowgrp, 0)

      pltpu.sync_copy(outb, out_hbm.at[pl.ds(base, R)])
      return carry

    lax.fori_loop(0, n_chunks, chunk, 0)

  return sc_kernel


def kernel(rate_idx, genre_idx, director_idx, actors_idx,
           table_rate, table_genre, table_director, table_actor):
  B = rate_idx.shape[0]
  n_chunks = (B // NW) // R
  idx_all = jnp.concatenate(
      [rate_idx[None, :], genre_idx.T, director_idx.T, actors_idx.T],
      axis=0).astype(jnp.int32)
  idx_all = idx_all.reshape(NP, NW, n_chunks, R).transpose(1, 2, 0, 3)
  return _build(B)(idx_all, table_rate, table_genre, table_director,
                   table_actor)


# speedup vs baseline: 2.0697x; 1.0550x over previous
import functools

import jax
import jax.numpy as jnp
from jax import lax
from jax.experimental import pallas as pl
from jax.experimental.pallas import tpu as pltpu
from jax.experimental.pallas import tpu_sc as plsc

D = 32
L = 16
NC, NS = 2, 16
NW = NC * NS
R = 32

K_GENRE, K_DIR, K_ACTOR = 8, 5, 20
SEG_RATE = 0
SEG_GENRE = R
SEG_DIR = SEG_GENRE + R * K_GENRE
SEG_ACTOR = SEG_DIR + R * K_DIR
NPR = SEG_ACTOR + R * K_ACTOR
POOLED = ((0, SEG_GENRE, K_GENRE), (1, SEG_DIR, K_DIR), (2, SEG_ACTOR, K_ACTOR))


@functools.lru_cache(maxsize=None)
def _build(B: int):
  rows_per_w = B // NW
  n_chunks = rows_per_w // R
  assert n_chunks % 2 == 0
  mesh = plsc.VectorSubcoreMesh(core_axis_name="c", subcore_axis_name="s",
                                num_cores=NC, num_subcores=NS)

  @functools.partial(
      pl.kernel,
      out_type=jax.ShapeDtypeStruct((B, 4 * D), jnp.float32),
      mesh=mesh,
      compiler_params=pltpu.CompilerParams(use_tc_tiling_on_sc=False,
                                           needs_layout_passes=False),
      scratch_types=[
          pltpu.VMEM((2, NPR, D), jnp.float32),
          pltpu.VMEM((2, R), jnp.int32),
          pltpu.VMEM((2, R * K_GENRE), jnp.int32),
          pltpu.VMEM((2, R * K_DIR), jnp.int32),
          pltpu.VMEM((2, R * K_ACTOR), jnp.int32),
          pltpu.VMEM((2, 3, R), jnp.float32),
          pltpu.VMEM((2, 3, R), jnp.float32),
          pltpu.VMEM((3, D), jnp.float32),
          pltpu.VMEM((2, R, 4 * D), jnp.float32),
          pltpu.SemaphoreType.DMA,
          pltpu.SemaphoreType.DMA,
          pltpu.SemaphoreType.DMA,
          pltpu.SemaphoreType.DMA,
          pltpu.SemaphoreType.DMA,
          pltpu.SemaphoreType.DMA,
      ],
  )
  def sc_kernel(ri_hbm, gi_hbm, di_hbm, ai_hbm,
                tr_hbm, tg_hbm, td_hbm, ta_hbm, out_hbm,
                gbuf, idxr, idxg, idxd, idxa, ab, ab2, t0, outb,
                g0, g1, i0, i1, o0, o1):
    wid = lax.axis_index("c") * NS + lax.axis_index("s")
    gsem = (g0, g1)
    isem = (i0, i1)
    osem = (o0, o1)

    pltpu.sync_copy(tg_hbm.at[pl.ds(0, 1)], t0.at[pl.ds(0, 1)])
    pltpu.sync_copy(td_hbm.at[pl.ds(0, 1)], t0.at[pl.ds(1, 1)])
    pltpu.sync_copy(ta_hbm.at[pl.ds(0, 1)], t0.at[pl.ds(2, 1)])
    t0v = [[t0[fi, pl.ds(h * L, L)] for h in range(2)] for fi in range(3)]

    idx_srcs = ((ri_hbm, idxr, 1), (gi_hbm, idxg, K_GENRE),
                (di_hbm, idxd, K_DIR), (ai_hbm, idxa, K_ACTOR))

    def stage_idx(c, s, sem):
      base = wid * rows_per_w + c * R
      for src, dst, k in idx_srcs:
        pltpu.async_copy(src.at[pl.ds(base * k, R * k)], dst.at[s], sem)

    def wait_idx(s, sem):
      for src, dst, k in idx_srcs:
        pltpu.make_async_copy(src.at[pl.ds(0, R * k)], dst.at[s], sem).wait()

    gplan = (
        (tr_hbm, idxr, SEG_RATE, 1, R),
        (tg_hbm, idxg, SEG_GENRE, K_GENRE, 128 // K_GENRE),
        (td_hbm, idxd, SEG_DIR, K_DIR, 16),
        (ta_hbm, idxa, SEG_ACTOR, K_ACTOR, 6),
    )

    def fire_gathers(s, sem):
      for tbl, isrc, seg, k, rows in gplan:
        r0 = 0
        while r0 < R:
          n = min(rows, R - r0)
          iref = isrc.at[s, pl.ds(r0 * k, n * k)]
          pltpu.async_copy(tbl.at[iref],
                           gbuf.at[s, pl.ds(seg + r0 * k, n * k)], sem)
          r0 += n

    def wait_gathers(s, sem):
      pltpu.make_async_copy(ta_hbm.at[pl.ds(0, NPR)], gbuf.at[s], sem).wait()

    viota = lax.iota(jnp.int32, L)

    def phase(c, s):
      o = 1 - s
      base = wid * rows_per_w + c * R

      for fi, seg, k in POOLED:
        isrc = (idxg, idxd, idxa)[fi]
        for g in range(R // L):
          rows = (viota + g * L) * k
          cnt = jnp.zeros((L,), jnp.float32)
          for j in range(k):
            v = plsc.load_gather(isrc.at[s], [rows + j])
            cnt = cnt + jnp.where(v > 0, jnp.float32(1.0), jnp.float32(0.0))
          a = jnp.float32(1.0) / (cnt + jnp.float32(1e-8))
          ab[s, fi, pl.ds(g * L, L)] = a
          ab2[s, fi, pl.ds(g * L, L)] = (jnp.float32(k) - cnt) * a

      @pl.when(c + 1 < n_chunks)
      def _():
        wait_idx(o, isem[o])
        fire_gathers(o, gsem[o])

      wait_gathers(s, gsem[s])

      @pl.when(c + 2 < n_chunks)
      def _():
        stage_idx(c + 2, s, isem[s])

      @pl.when(c >= 2)
      def _():
        pltpu.make_async_copy(out_hbm.at[pl.ds(0, R)], outb.at[s],
                              osem[s]).wait()

      def row(r, carry):
        rsp = jnp.full((L,), r, jnp.int32)
        for h in range(2):
          sl = pl.ds(h * L, L)
          outb[s, r, pl.ds(h * L, L)] = gbuf[s, SEG_RATE + r, sl]
        for fi, seg, k in POOLED:
          av = plsc.load_gather(ab.at[s, fi], [rsp])
          bv = plsc.load_gather(ab2.at[s, fi], [rsp])
          for h in range(2):
            sl = pl.ds(h * L, L)
            acc = gbuf[s, seg + r * k, sl]
            for j in range(1, k):
              acc = acc + gbuf[s, seg + r * k + j, sl]
            outb[s, r, pl.ds((fi + 1) * D + h * L, L)] = (
                acc * av - t0v[fi][h] * bv)
        return carry

      lax.fori_loop(0, R, row, 0)

      pltpu.async_copy(outb.at[s], out_hbm.at[pl.ds(base, R)], osem[s])

    stage_idx(0, 0, isem[0])
    wait_idx(0, isem[0])
    fire_gathers(0, gsem[0])
    stage_idx(1, 1, isem[1])

    def pair(i, carry):
      phase(2 * i, 0)
      phase(2 * i + 1, 1)
      return carry

    lax.fori_loop(0, n_chunks // 2, pair, 0)

    for s in range(2):
      pltpu.make_async_copy(out_hbm.at[pl.ds(0, R)], outb.at[s],
                            osem[s]).wait()

  return sc_kernel


def kernel(rate_idx, genre_idx, director_idx, actors_idx,
           table_rate, table_genre, table_director, table_actor):
  B = rate_idx.shape[0]
  return _build(B)(rate_idx.astype(jnp.int32).reshape(-1),
                   genre_idx.astype(jnp.int32).reshape(-1),
                   director_idx.astype(jnp.int32).reshape(-1),
                   actors_idx.astype(jnp.int32).reshape(-1),
                   table_rate, table_genre, table_director, table_actor)
